# baseline (device time: 94408 ns/iter reference)
import jax
import jax.numpy as jnp
from jax import lax
from jax.experimental import pallas as pl
from jax.experimental.pallas import tpu as pltpu

N_DEV = 4


def kernel(x, w_mat, scale_x, scale_w):
    m_per, k = x.shape
    _, n_per = w_mat.shape
    q = m_per // 4

    def body(x_ref, w_ref, sx_ref, sw_ref, out_ref,
             x8_ref, w8_ref, comm_ref, send_sems, recv_sems):
        my = lax.axis_index("i")
        left = lax.rem(my + N_DEV - 1, N_DEV)
        right = lax.rem(my + 1, N_DEV)
        opp = lax.rem(my + 2, N_DEV)

        barrier = pltpu.get_barrier_semaphore()
        pl.semaphore_signal(barrier, inc=1, device_id=(left,),
                            device_id_type=pl.DeviceIdType.MESH)
        pl.semaphore_signal(barrier, inc=1, device_id=(right,),
                            device_id_type=pl.DeviceIdType.MESH)

        x8_ref[...] = x_ref[...].astype(jnp.float8_e5m2)
        w8_ref[...] = w_ref[...].astype(jnp.float8_e5m2)

        pl.semaphore_wait(barrier, 2)

        send_r = pltpu.make_async_remote_copy(
            src_ref=x8_ref, dst_ref=comm_ref.at[0],
            send_sem=send_sems.at[0], recv_sem=recv_sems.at[0],
            device_id=(right,), device_id_type=pl.DeviceIdType.MESH)
        send_r.start()
        send_l = pltpu.make_async_remote_copy(
            src_ref=x8_ref, dst_ref=comm_ref.at[1],
            send_sem=send_sems.at[1], recv_sem=recv_sems.at[1],
            device_id=(left,), device_id_type=pl.DeviceIdType.MESH)
        send_l.start()

        s = sx_ref[0] * sw_ref[0]

        def mm_store(row0, chunk):
            acc = lax.dot_general(chunk, w8_ref[...],
                                  (((1,), (0,)), ((), ())),
                                  preferred_element_type=jnp.float32)
            y = acc * s
            out_ref[pl.ds(row0, chunk.shape[0]), :] = y * (1.0 / (1.0 + jnp.exp(-y)))

        mm_store(my * m_per, x8_ref[...])

        send_r.wait_recv()
        fwd_r = []
        for j in range(2):
            f = pltpu.make_async_remote_copy(
                src_ref=comm_ref.at[0, pl.ds(j * q, q), :],
                dst_ref=comm_ref.at[2, pl.ds(j * q, q), :],
                send_sem=send_sems.at[2 + j], recv_sem=recv_sems.at[2 + j],
                device_id=(right,), device_id_type=pl.DeviceIdType.MESH)
            f.start()
            fwd_r.append(f)
        send_l.wait_recv()
        fwd_l = []
        for j in range(2):
            f = pltpu.make_async_remote_copy(
                src_ref=comm_ref.at[1, pl.ds((2 + j) * q, q), :],
                dst_ref=comm_ref.at[2, pl.ds((2 + j) * q, q), :],
                send_sem=send_sems.at[4 + j], recv_sem=recv_sems.at[4 + j],
                device_id=(left,), device_id_type=pl.DeviceIdType.MESH)
            f.start()
            fwd_l.append(f)

        mm_store(left * m_per, comm_ref[0])
        mm_store(right * m_per, comm_ref[1])

        fwd_r[0].wait_recv()
        mm_store(opp * m_per + 0 * q, comm_ref[2, pl.ds(0 * q, q), :])
        fwd_l[0].wait_recv()
        mm_store(opp * m_per + 2 * q, comm_ref[2, pl.ds(2 * q, q), :])
        fwd_r[1].wait_recv()
        mm_store(opp * m_per + 1 * q, comm_ref[2, pl.ds(1 * q, q), :])
        fwd_l[1].wait_recv()
        mm_store(opp * m_per + 3 * q, comm_ref[2, pl.ds(3 * q, q), :])

        send_r.wait_send()
        send_l.wait_send()
        for f in fwd_r + fwd_l:
            f.wait_send()

    return pl.pallas_call(
        body,
        out_shape=jax.ShapeDtypeStruct((N_DEV * m_per, n_per), jnp.float32),
        in_specs=[
            pl.BlockSpec(memory_space=pltpu.VMEM),
            pl.BlockSpec(memory_space=pltpu.VMEM),
            pl.BlockSpec(memory_space=pltpu.SMEM),
            pl.BlockSpec(memory_space=pltpu.SMEM),
        ],
        out_specs=pl.BlockSpec(memory_space=pltpu.VMEM),
        scratch_shapes=[
            pltpu.VMEM((m_per, k), jnp.float8_e5m2),
            pltpu.VMEM((k, n_per), jnp.float8_e5m2),
            pltpu.VMEM((3, m_per, k), jnp.float8_e5m2),
            pltpu.SemaphoreType.DMA((6,)),
            pltpu.SemaphoreType.DMA((6,)),
        ],
        compiler_params=pltpu.CompilerParams(
            collective_id=0, vmem_limit_bytes=60 * 1024 * 1024),
    )(x, w_mat, scale_x, scale_w)


# device time: 85489 ns/iter; 1.1043x vs baseline; 1.1043x over previous
import jax
import jax.numpy as jnp
from jax import lax
from jax.experimental import pallas as pl
from jax.experimental.pallas import tpu as pltpu

N_DEV = 4
N_STRIPES = 4


def kernel(x, w_mat, scale_x, scale_w):
    m_per, k = x.shape
    _, n_per = w_mat.shape
    sr = m_per // N_STRIPES

    def body(x_hbm, w_hbm, sx_ref, sw_ref, out_ref,
             x32_ref, w32_ref, x8_ref, w8_ref, comm_ref,
             fill_sems, send_sems, recv_sems):
        my = lax.axis_index("i")
        left = lax.rem(my + N_DEV - 1, N_DEV)
        right = lax.rem(my + 1, N_DEV)
        opp = lax.rem(my + 2, N_DEV)

        barrier = pltpu.get_barrier_semaphore()
        pl.semaphore_signal(barrier, inc=1, device_id=(left,),
                            device_id_type=pl.DeviceIdType.MESH)
        pl.semaphore_signal(barrier, inc=1, device_id=(right,),
                            device_id_type=pl.DeviceIdType.MESH)

        fills = []
        for j in range(2):
            f = pltpu.make_async_copy(
                x_hbm.at[pl.ds(j * sr, sr), :], x32_ref.at[j],
                fill_sems.at[j])
            f.start()
            fills.append(f)

        pl.semaphore_wait(barrier, 2)

        sends_r, sends_l = [], []
        for j in range(N_STRIPES):
            slot = j % 2
            fills[j].wait()
            x8_ref[pl.ds(j * sr, sr), :] = x32_ref[slot].astype(jnp.float8_e5m2)
            if j + 2 < N_STRIPES:
                f = pltpu.make_async_copy(
                    x_hbm.at[pl.ds((j + 2) * sr, sr), :], x32_ref.at[slot],
                    fill_sems.at[slot])
                f.start()
                fills.append(f)
            s = pltpu.make_async_remote_copy(
                src_ref=x8_ref.at[pl.ds(j * sr, sr), :],
                dst_ref=comm_ref.at[0, pl.ds(j * sr, sr), :],
                send_sem=send_sems.at[j], recv_sem=recv_sems.at[j],
                device_id=(right,), device_id_type=pl.DeviceIdType.MESH)
            s.start()
            sends_r.append(s)
            s = pltpu.make_async_remote_copy(
                src_ref=x8_ref.at[pl.ds(j * sr, sr), :],
                dst_ref=comm_ref.at[1, pl.ds(j * sr, sr), :],
                send_sem=send_sems.at[N_STRIPES + j],
                recv_sem=recv_sems.at[N_STRIPES + j],
                device_id=(left,), device_id_type=pl.DeviceIdType.MESH)
            s.start()
            sends_l.append(s)

        wf = pltpu.make_async_copy(w_hbm, w32_ref, fill_sems.at[2])
        wf.start()
        wf.wait()
        w8_ref[...] = w32_ref[...].astype(jnp.float8_e5m2)

        sc = sx_ref[0] * sw_ref[0]

        def mm_store(row0, chunk):
            acc = lax.dot_general(chunk, w8_ref[...],
                                  (((1,), (0,)), ((), ())),
                                  preferred_element_type=jnp.float32)
            y = acc * sc
            out_ref[pl.ds(row0, chunk.shape[0]), :] = y * (1.0 / (1.0 + jnp.exp(-y)))

        mm_store(my * m_per, x8_ref[...])

        fwds = []
        for j in range(2):
            sends_r[j].wait_recv()
            f = pltpu.make_async_remote_copy(
                src_ref=comm_ref.at[0, pl.ds(j * sr, sr), :],
                dst_ref=comm_ref.at[2, pl.ds(j * sr, sr), :],
                send_sem=send_sems.at[8 + j], recv_sem=recv_sems.at[8 + j],
                device_id=(right,), device_id_type=pl.DeviceIdType.MESH)
            f.start()
            fwds.append(f)
        for j in range(2, 4):
            sends_l[j].wait_recv()
            f = pltpu.make_async_remote_copy(
                src_ref=comm_ref.at[1, pl.ds(j * sr, sr), :],
                dst_ref=comm_ref.at[2, pl.ds(j * sr, sr), :],
                send_sem=send_sems.at[8 + j], recv_sem=recv_sems.at[8 + j],
                device_id=(left,), device_id_type=pl.DeviceIdType.MESH)
            f.start()
            fwds.append(f)

        sends_r[2].wait_recv()
        sends_r[3].wait_recv()
        mm_store(left * m_per, comm_ref[0])
        sends_l[0].wait_recv()
        sends_l[1].wait_recv()
        mm_store(right * m_per, comm_ref[1])

        for j in (0, 2, 1, 3):
            fwds[j].wait_recv()
            mm_store(opp * m_per + j * sr, comm_ref[2, pl.ds(j * sr, sr), :])

        for s in sends_r + sends_l + fwds:
            s.wait_send()

    return pl.pallas_call(
        body,
        out_shape=jax.ShapeDtypeStruct((N_DEV * m_per, n_per), jnp.float32),
        in_specs=[
            pl.BlockSpec(memory_space=pl.ANY),
            pl.BlockSpec(memory_space=pl.ANY),
            pl.BlockSpec(memory_space=pltpu.SMEM),
            pl.BlockSpec(memory_space=pltpu.SMEM),
        ],
        out_specs=pl.BlockSpec(memory_space=pltpu.VMEM),
        scratch_shapes=[
            pltpu.VMEM((2, sr, k), jnp.float32),
            pltpu.VMEM((k, n_per), jnp.float32),
            pltpu.VMEM((m_per, k), jnp.float8_e5m2),
            pltpu.VMEM((k, n_per), jnp.float8_e5m2),
            pltpu.VMEM((3, m_per, k), jnp.float8_e5m2),
            pltpu.SemaphoreType.DMA((3,)),
            pltpu.SemaphoreType.DMA((12,)),
            pltpu.SemaphoreType.DMA((12,)),
        ],
        compiler_params=pltpu.CompilerParams(
            collective_id=0, vmem_limit_bytes=60 * 1024 * 1024),
    )(x, w_mat, scale_x, scale_w)


# device time: 81814 ns/iter; 1.1539x vs baseline; 1.0449x over previous
import jax
import jax.numpy as jnp
from jax import lax
from jax.experimental import pallas as pl
from jax.experimental.pallas import tpu as pltpu

N_DEV = 4

R_PIECES = [(0, 128), (128, 128), (256, 256), (512, 512)]
L_PIECES = [(896, 128), (768, 128), (512, 256), (0, 512)]
FILLS = [(0, 128), (896, 128), (128, 128), (768, 128), (256, 256), (512, 256)]
FWD_R = [(256, 256), (128, 128), (0, 128)]
FWD_L = [(512, 256), (768, 128), (896, 128)]


def kernel(x, w_mat, scale_x, scale_w):
    m_per, k = x.shape
    _, n_per = w_mat.shape

    def body(x_hbm, w_hbm, sx_ref, sw_ref, out_hbm,
             x32_ref, w32_ref, x8_ref, w8_ref, comm_ref, y_ref,
             fill_sems, send_sems, recv_sems, out_sems):
        my = lax.axis_index("i")
        left = lax.rem(my + N_DEV - 1, N_DEV)
        right = lax.rem(my + 1, N_DEV)
        opp = lax.rem(my + 2, N_DEV)

        barrier = pltpu.get_barrier_semaphore()
        pl.semaphore_signal(barrier, inc=1, device_id=(left,),
                            device_id_type=pl.DeviceIdType.MESH)
        pl.semaphore_signal(barrier, inc=1, device_id=(right,),
                            device_id_type=pl.DeviceIdType.MESH)

        fills = []
        for i, (r0, nr) in enumerate(FILLS):
            f = pltpu.make_async_copy(
                x_hbm.at[pl.ds(r0, nr), :], x32_ref.at[pl.ds(r0, nr), :],
                fill_sems.at[i])
            f.start()
            fills.append(f)
        wf = pltpu.make_async_copy(w_hbm, w32_ref, fill_sems.at[len(FILLS)])
        wf.start()

        pl.semaphore_wait(barrier, 2)

        def cast(i):
            r0, nr = FILLS[i]
            fills[i].wait()
            x8_ref[pl.ds(r0, nr), :] = x32_ref[pl.ds(r0, nr), :].astype(
                jnp.float8_e5m2)

        def send(pieces, idx, slot, dev, sem_base):
            r0, nr = pieces[idx]
            s = pltpu.make_async_remote_copy(
                src_ref=x8_ref.at[pl.ds(r0, nr), :],
                dst_ref=comm_ref.at[slot, pl.ds(r0, nr), :],
                send_sem=send_sems.at[sem_base + idx],
                recv_sem=recv_sems.at[sem_base + idx],
                device_id=dev, device_id_type=pl.DeviceIdType.MESH)
            s.start()
            return s

        sends_r, sends_l = [None] * 4, [None] * 4
        cast(0); sends_r[0] = send(R_PIECES, 0, 0, (right,), 0)
        cast(1); sends_l[0] = send(L_PIECES, 0, 1, (left,), 4)
        cast(2); sends_r[1] = send(R_PIECES, 1, 0, (right,), 0)
        cast(3); sends_l[1] = send(L_PIECES, 1, 1, (left,), 4)
        cast(4); sends_r[2] = send(R_PIECES, 2, 0, (right,), 0)
        cast(5)
        sends_r[3] = send(R_PIECES, 3, 0, (right,), 0)
        sends_l[2] = send(L_PIECES, 2, 1, (left,), 4)
        sends_l[3] = send(L_PIECES, 3, 1, (left,), 4)

        wf.wait()
        w8_ref[...] = w32_ref[...].astype(jnp.float8_e5m2)

        sc = sx_ref[0] * sw_ref[0]
        out_dmas = []

        def mm_store(row0, chunk, slot, srow):
            acc = lax.dot_general(chunk, w8_ref[...],
                                  (((1,), (0,)), ((), ())),
                                  preferred_element_type=jnp.float32)
            y = acc * sc
            nrow = chunk.shape[0]
            y_ref[slot, pl.ds(srow, nrow), :] = y * (1.0 / (1.0 + jnp.exp(-y)))
            d = pltpu.make_async_copy(
                y_ref.at[slot, pl.ds(srow, nrow), :],
                out_hbm.at[pl.ds(row0, nrow), :],
                out_sems.at[len(out_dmas)])
            d.start()
            out_dmas.append(d)

        mm_store(my * m_per, x8_ref[...], 0, 0)

        fwds = []
        for fi, (r0, nr) in enumerate(FWD_R):
            sends_r[2 - fi].wait_recv()
            f = pltpu.make_async_remote_copy(
                src_ref=comm_ref.at[0, pl.ds(r0, nr), :],
                dst_ref=comm_ref.at[2, pl.ds(r0, nr), :],
                send_sem=send_sems.at[8 + fi], recv_sem=recv_sems.at[8 + fi],
                device_id=(right,), device_id_type=pl.DeviceIdType.MESH)
            f.start()
            fwds.append(f)
        for fi, (r0, nr) in enumerate(FWD_L):
            sends_l[2 - fi].wait_recv()
            f = pltpu.make_async_remote_copy(
                src_ref=comm_ref.at[1, pl.ds(r0, nr), :],
                dst_ref=comm_ref.at[2, pl.ds(r0, nr), :],
                send_sem=send_sems.at[11 + fi], recv_sem=recv_sems.at[11 + fi],
                device_id=(left,), device_id_type=pl.DeviceIdType.MESH)
            f.start()
            fwds.append(f)

        sends_r[3].wait_recv()
        mm_store(left * m_per, comm_ref[0], 1, 0)
        sends_l[3].wait_recv()
        mm_store(right * m_per, comm_ref[1], 2, 0)

        order = [0, 3, 1, 4, 2, 5]
        rowmap = FWD_R + FWD_L
        for oi in order:
            fwds[oi].wait_recv()
            r0, nr = rowmap[oi]
            mm_store(opp * m_per + r0, comm_ref[2, pl.ds(r0, nr), :], 3, r0)

        for s in sends_r + sends_l + fwds:
            s.wait_send()
        for d in out_dmas:
            d.wait()

    return pl.pallas_call(
        body,
        out_shape=jax.ShapeDtypeStruct((N_DEV * m_per, n_per), jnp.float32),
        in_specs=[
            pl.BlockSpec(memory_space=pl.ANY),
            pl.BlockSpec(memory_space=pl.ANY),
            pl.BlockSpec(memory_space=pltpu.SMEM),
            pl.BlockSpec(memory_space=pltpu.SMEM),
        ],
        out_specs=pl.BlockSpec(memory_space=pl.ANY),
        scratch_shapes=[
            pltpu.VMEM((m_per, k), jnp.float32),
            pltpu.VMEM((k, n_per), jnp.float32),
            pltpu.VMEM((m_per, k), jnp.float8_e5m2),
            pltpu.VMEM((k, n_per), jnp.float8_e5m2),
            pltpu.VMEM((3, m_per, k), jnp.float8_e5m2),
            pltpu.VMEM((4, m_per, n_per), jnp.float32),
            pltpu.SemaphoreType.DMA((7,)),
            pltpu.SemaphoreType.DMA((14,)),
            pltpu.SemaphoreType.DMA((14,)),
            pltpu.SemaphoreType.DMA((12,)),
        ],
        compiler_params=pltpu.CompilerParams(
            collective_id=0, vmem_limit_bytes=60 * 1024 * 1024),
    )(x, w_mat, scale_x, scale_w)


# device time: 81205 ns/iter; 1.1626x vs baseline; 1.0075x over previous
import jax
import jax.numpy as jnp
from jax import lax
from jax.experimental import pallas as pl
from jax.experimental.pallas import tpu as pltpu

N_DEV = 4

R_PIECES = [(0, 64), (64, 64), (128, 128), (256, 256), (512, 512)]
L_PIECES = [(960, 64), (896, 64), (768, 128), (512, 256), (0, 512)]
FILLS = [(0, 64), (960, 64), (64, 64), (896, 64), (128, 128),
         (768, 128), (256, 256), (512, 256)]
FWD_R = [(256, 256), (128, 128), (64, 64), (0, 64)]
FWD_L = [(512, 256), (768, 128), (896, 64), (960, 64)]


def kernel(x, w_mat, scale_x, scale_w):
    m_per, k = x.shape
    _, n_per = w_mat.shape

    def body(x_hbm, w_hbm, sx_ref, sw_ref, out_hbm,
             x32_ref, w32_ref, x8_ref, w8_ref, comm_ref, y_ref,
             fill_sems, send_sems, recv_sems, out_sems):
        my = lax.axis_index("i")
        left = lax.rem(my + N_DEV - 1, N_DEV)
        right = lax.rem(my + 1, N_DEV)
        opp = lax.rem(my + 2, N_DEV)

        barrier = pltpu.get_barrier_semaphore()
        pl.semaphore_signal(barrier, inc=1, device_id=(left,),
                            device_id_type=pl.DeviceIdType.MESH)
        pl.semaphore_signal(barrier, inc=1, device_id=(right,),
                            device_id_type=pl.DeviceIdType.MESH)

        fills = []
        for i, (r0, nr) in enumerate(FILLS):
            f = pltpu.make_async_copy(
                x_hbm.at[pl.ds(r0, nr), :], x32_ref.at[pl.ds(r0, nr), :],
                fill_sems.at[i])
            f.start()
            fills.append(f)

        pl.semaphore_wait(barrier, 2)

        def cast(i):
            r0, nr = FILLS[i]
            fills[i].wait()
            x8_ref[pl.ds(r0, nr), :] = x32_ref[pl.ds(r0, nr), :].astype(
                jnp.float8_e5m2)

        def send(pieces, idx, slot, dev, sem_base):
            r0, nr = pieces[idx]
            s = pltpu.make_async_remote_copy(
                src_ref=x8_ref.at[pl.ds(r0, nr), :],
                dst_ref=comm_ref.at[slot, pl.ds(r0, nr), :],
                send_sem=send_sems.at[sem_base + idx],
                recv_sem=recv_sems.at[sem_base + idx],
                device_id=dev, device_id_type=pl.DeviceIdType.MESH)
            s.start()
            return s

        sends_r, sends_l = [None] * 5, [None] * 5
        cast(0); sends_r[0] = send(R_PIECES, 0, 0, (right,), 0)
        cast(1); sends_l[0] = send(L_PIECES, 0, 1, (left,), 5)
        wf = pltpu.make_async_copy(w_hbm, w32_ref, fill_sems.at[len(FILLS)])
        wf.start()
        cast(2); sends_r[1] = send(R_PIECES, 1, 0, (right,), 0)
        cast(3); sends_l[1] = send(L_PIECES, 1, 1, (left,), 5)
        cast(4); sends_r[2] = send(R_PIECES, 2, 0, (right,), 0)
        cast(5); sends_l[2] = send(L_PIECES, 2, 1, (left,), 5)
        cast(6); sends_r[3] = send(R_PIECES, 3, 0, (right,), 0)
        cast(7)
        sends_r[4] = send(R_PIECES, 4, 0, (right,), 0)
        sends_l[3] = send(L_PIECES, 3, 1, (left,), 5)
        sends_l[4] = send(L_PIECES, 4, 1, (left,), 5)

        wf.wait()
        w8_ref[...] = w32_ref[...].astype(jnp.float8_e5m2)

        sc = sx_ref[0] * sw_ref[0]
        out_dmas = []

        def mm_store(row0, chunk, slot, srow):
            acc = lax.dot_general(chunk, w8_ref[...],
                                  (((1,), (0,)), ((), ())),
                                  preferred_element_type=jnp.float32)
            y = acc * sc
            nrow = chunk.shape[0]
            y_ref[slot, pl.ds(srow, nrow), :] = y * (1.0 / (1.0 + jnp.exp(-y)))
            d = pltpu.make_async_copy(
                y_ref.at[slot, pl.ds(srow, nrow), :],
                out_hbm.at[pl.ds(row0, nrow), :],
                out_sems.at[len(out_dmas)])
            d.start()
            out_dmas.append(d)

        mm_store(my * m_per, x8_ref[...], 0, 0)

        fwds = []
        for fi, (r0, nr) in enumerate(FWD_R):
            sends_r[3 - fi].wait_recv()
            f = pltpu.make_async_remote_copy(
                src_ref=comm_ref.at[0, pl.ds(r0, nr), :],
                dst_ref=comm_ref.at[2, pl.ds(r0, nr), :],
                send_sem=send_sems.at[10 + fi], recv_sem=recv_sems.at[10 + fi],
                device_id=(right,), device_id_type=pl.DeviceIdType.MESH)
            f.start()
            fwds.append(f)
        for fi, (r0, nr) in enumerate(FWD_L):
            sends_l[3 - fi].wait_recv()
            f = pltpu.make_async_remote_copy(
                src_ref=comm_ref.at[1, pl.ds(r0, nr), :],
                dst_ref=comm_ref.at[2, pl.ds(r0, nr), :],
                send_sem=send_sems.at[14 + fi], recv_sem=recv_sems.at[14 + fi],
                device_id=(left,), device_id_type=pl.DeviceIdType.MESH)
            f.start()
            fwds.append(f)

        sends_r[4].wait_recv()
        mm_store(left * m_per, comm_ref[0], 1, 0)
        sends_l[4].wait_recv()
        mm_store(right * m_per, comm_ref[1], 2, 0)

        order = [0, 4, 1, 5, 2, 6, 3, 7]
        rowmap = FWD_R + FWD_L
        for oi in order:
            fwds[oi].wait_recv()
            r0, nr = rowmap[oi]
            mm_store(opp * m_per + r0, comm_ref[2, pl.ds(r0, nr), :], 3, r0)

        for s in sends_r + sends_l + fwds:
            s.wait_send()
        for d in out_dmas:
            d.wait()

    return pl.pallas_call(
        body,
        out_shape=jax.ShapeDtypeStruct((N_DEV * m_per, n_per), jnp.float32),
        in_specs=[
            pl.BlockSpec(memory_space=pl.ANY),
            pl.BlockSpec(memory_space=pl.ANY),
            pl.BlockSpec(memory_space=pltpu.SMEM),
            pl.BlockSpec(memory_space=pltpu.SMEM),
        ],
        out_specs=pl.BlockSpec(memory_space=pl.ANY),
        scratch_shapes=[
            pltpu.VMEM((m_per, k), jnp.float32),
            pltpu.VMEM((k, n_per), jnp.float32),
            pltpu.VMEM((m_per, k), jnp.float8_e5m2),
            pltpu.VMEM((k, n_per), jnp.float8_e5m2),
            pltpu.VMEM((3, m_per, k), jnp.float8_e5m2),
            pltpu.VMEM((4, m_per, n_per), jnp.float32),
            pltpu.SemaphoreType.DMA((9,)),
            pltpu.SemaphoreType.DMA((18,)),
            pltpu.SemaphoreType.DMA((18,)),
            pltpu.SemaphoreType.DMA((12,)),
        ],
        compiler_params=pltpu.CompilerParams(
            collective_id=0, vmem_limit_bytes=60 * 1024 * 1024),
    )(x, w_mat, scale_x, scale_w)
